# baseline (speedup 1.0000x reference)
import functools
import math

import jax
import jax.numpy as jnp
from jax import lax
from jax.experimental import pallas as pl
from jax.experimental.pallas import tpu as pltpu
from jax.experimental.pallas import tpu_sc as plsc

_LANES = 16
_CHUNK = 128


def _emb_body(n_chunks, chunk, D, n_per_w,
              idx_hbm, table_hbm, out_hbm, idx_v, gbuf, sbuf, gsem):
    wid = lax.axis_index("s") * 2 + lax.axis_index("c")
    pltpu.sync_copy(idx_hbm.at[pl.ds(wid * n_chunks, n_chunks)], idx_v)
    row_base = wid * n_per_w
    scale = jnp.full((_LANES,), math.sqrt(D), dtype=jnp.float32)

    def chunk_body(g, carry):
        pltpu.async_copy(table_hbm.at[idx_v.at[g]], gbuf, gsem).wait()

        def row_body(r, c2):
            for c in range(D // _LANES):
                sbuf[r, pl.ds(c * _LANES, _LANES)] = (
                    gbuf[r, pl.ds(c * _LANES, _LANES)] * scale)
            return c2

        lax.fori_loop(0, chunk, row_body, 0, unroll=2)
        pltpu.sync_copy(sbuf, out_hbm.at[pl.ds(row_base + g * chunk, chunk)])
        return carry

    lax.fori_loop(0, n_chunks, chunk_body, 0)


def kernel(input_x, table):
    B0, S = input_x.shape
    V, D = table.shape
    B = B0 * S
    n_workers = 32
    n_per_w = B // n_workers
    n_chunks = n_per_w // _CHUNK
    idx2d = input_x.reshape(B // _CHUNK, _CHUNK).astype(jnp.int32)

    mesh = plsc.VectorSubcoreMesh(core_axis_name="c", subcore_axis_name="s")
    emb = pl.kernel(
        functools.partial(_emb_body, n_chunks, _CHUNK, D, n_per_w),
        mesh=mesh,
        out_type=jax.ShapeDtypeStruct((B, D), jnp.float32),
        scratch_types=[
            pltpu.VMEM((n_chunks, _CHUNK), jnp.int32),
            pltpu.VMEM((_CHUNK, D), jnp.float32),
            pltpu.VMEM((_CHUNK, D), jnp.float32),
            pltpu.SemaphoreType.DMA,
        ],
        compiler_params=pltpu.CompilerParams(use_tc_tiling_on_sc=False),
    )
    out = emb(idx2d, table)
    return out.reshape(B0, S, D)


# speedup vs baseline: 1.1777x; 1.1777x over previous
import functools
import math

import jax
import jax.numpy as jnp
from jax import lax
from jax.experimental import pallas as pl
from jax.experimental.pallas import tpu as pltpu
from jax.experimental.pallas import tpu_sc as plsc

_LANES = 16
_CHUNK = 128
_NBUF = 4


def _emb_body(n_chunks, chunk, D, n_per_w,
              idx_hbm, table_hbm, out_hbm, idx_v, gbufs, sbufs, gsems, ssems):
    wid = lax.axis_index("s") * 2 + lax.axis_index("c")
    pltpu.sync_copy(idx_hbm.at[pl.ds(wid * n_chunks, n_chunks)], idx_v)
    row_base = wid * n_per_w
    scale = jnp.full((_LANES,), math.sqrt(D), dtype=jnp.float32)
    n_rounds = n_chunks // _NBUF

    def start_gather(g, b):
        pltpu.async_copy(table_hbm.at[idx_v.at[g]], gbufs[b], gsems[b])

    def wait_gather(g, b):
        pltpu.make_async_copy(
            table_hbm.at[idx_v.at[g]], gbufs[b], gsems[b]).wait()

    def start_scatter(g, b):
        pltpu.async_copy(
            sbufs[b], out_hbm.at[pl.ds(row_base + g * chunk, chunk)], ssems[b])

    def wait_scatter(g, b):
        pltpu.make_async_copy(
            sbufs[b], out_hbm.at[pl.ds(row_base + g * chunk, chunk)],
            ssems[b]).wait()

    for b in range(_NBUF):
        start_gather(b, b)

    def round_body(t, carry):
        for b in range(_NBUF):
            g = t * _NBUF + b
            wait_gather(g, b)

            @pl.when(t > 0)
            def _():
                wait_scatter(g - _NBUF, b)

            def row_body(r, c2):
                for c in range(D // _LANES):
                    sbufs[b][r, pl.ds(c * _LANES, _LANES)] = (
                        gbufs[b][r, pl.ds(c * _LANES, _LANES)] * scale)
                return c2

            lax.fori_loop(0, chunk, row_body, 0, unroll=2)

            @pl.when(t < n_rounds - 1)
            def _():
                start_gather(g + _NBUF, b)

            start_scatter(g, b)
        return carry

    lax.fori_loop(0, n_rounds, round_body, 0)
    for b in range(_NBUF):
        wait_scatter((n_rounds - 1) * _NBUF + b, b)


def kernel(input_x, table):
    B0, S = input_x.shape
    V, D = table.shape
    B = B0 * S
    n_workers = 32
    n_per_w = B // n_workers
    n_chunks = n_per_w // _CHUNK
    idx2d = input_x.reshape(B // _CHUNK, _CHUNK).astype(jnp.int32)

    mesh = plsc.VectorSubcoreMesh(core_axis_name="c", subcore_axis_name="s")
    emb = pl.kernel(
        functools.partial(_emb_body, n_chunks, _CHUNK, D, n_per_w),
        mesh=mesh,
        out_type=jax.ShapeDtypeStruct((B, D), jnp.float32),
        scratch_types=[
            pltpu.VMEM((n_chunks, _CHUNK), jnp.int32),
            [pltpu.VMEM((_CHUNK, D), jnp.float32) for _ in range(_NBUF)],
            [pltpu.VMEM((_CHUNK, D), jnp.float32) for _ in range(_NBUF)],
            [pltpu.SemaphoreType.DMA for _ in range(_NBUF)],
            [pltpu.SemaphoreType.DMA for _ in range(_NBUF)],
        ],
        compiler_params=pltpu.CompilerParams(use_tc_tiling_on_sc=False),
    )
    out = emb(idx2d, table)
    return out.reshape(B0, S, D)
